# baseline (speedup 1.0000x reference)
import functools

import jax
import jax.numpy as jnp
from jax import lax
from jax.experimental import pallas as pl
from jax.experimental.pallas import tpu as pltpu
from jax.experimental.pallas import tpu_sc as plsc

_B = 16384
_D = 64
_SHARED = 16
_REST = _D - _SHARED
_CHUNK = 512


@functools.cache
def _build():
    try:
        nc = plsc.get_sparse_core_info().num_cores
    except Exception:
        nc = 2
    bpc = _B // nc
    nch = bpc // _CHUNK
    mesh = plsc.ScalarSubcoreMesh(axis_name="c")

    @functools.partial(
        pl.kernel,
        mesh=mesh,
        out_type=jax.ShapeDtypeStruct((_B, _D), jnp.float32),
        scratch_types=[
            pltpu.SMEM((_CHUNK,), jnp.int32),
            pltpu.VMEM_SHARED((bpc, _D), jnp.float32),
            pltpu.SemaphoreType.DMA,
            pltpu.SemaphoreType.DMA,
            pltpu.SemaphoreType.DMA,
        ],
    )
    def gather_kernel(x_hbm, table_hbm, shared_hbm, out_hbm,
                      idx_s, stage, sem_sh, sem_tb, sem_out):
        cid = lax.axis_index("c")
        base = cid * bpc

        for c in range(nch):
            pltpu.sync_copy(x_hbm.at[pl.ds(base + c * _CHUNK, _CHUNK)], idx_s)

            def row(i, carry):
                r = idx_s[i]
                o = c * _CHUNK + i
                pltpu.async_copy(
                    shared_hbm, stage.at[o, pl.ds(0, _SHARED)], sem_sh
                )
                pltpu.async_copy(
                    table_hbm.at[r, pl.ds(_SHARED, _REST)],
                    stage.at[o, pl.ds(_SHARED, _REST)],
                    sem_tb,
                )
                return carry

            lax.fori_loop(0, _CHUNK, row, 0, unroll=8)

        def drain(i, carry):
            pltpu.make_async_copy(
                shared_hbm, stage.at[i, pl.ds(0, _SHARED)], sem_sh
            ).wait()
            pltpu.make_async_copy(
                table_hbm.at[0, pl.ds(_SHARED, _REST)],
                stage.at[i, pl.ds(_SHARED, _REST)],
                sem_tb,
            ).wait()
            return carry

        lax.fori_loop(0, bpc, drain, 0, unroll=8)
        pltpu.async_copy(stage, out_hbm.at[pl.ds(base, bpc)], sem_out).wait()

    return gather_kernel


def kernel(X, table, shared_embed):
    return _build()(X, table, shared_embed.reshape(_SHARED))


# speedup vs baseline: 1.2995x; 1.2995x over previous
import functools

import jax
import jax.numpy as jnp
from jax import lax
from jax.experimental import pallas as pl
from jax.experimental.pallas import tpu as pltpu
from jax.experimental.pallas import tpu_sc as plsc
from jax._src.pallas import mpmd

_B = 16384
_D = 64
_SHARED = 16
_REST = _D - _SHARED
_SROWS = 6144
_CHUNK = 512


@functools.cache
def _build():
    try:
        info = plsc.get_sparse_core_info()
        nc, ns = info.num_cores, info.num_subcores
    except Exception:
        nc, ns = 2, 16
    nw = nc * ns
    bpc = _SROWS // nc
    nch = bpc // _CHUNK
    vrows = _B - _SROWS
    bpw = vrows // nw
    smesh = plsc.ScalarSubcoreMesh(axis_name="c")
    vmesh = plsc.VectorSubcoreMesh(core_axis_name="c", subcore_axis_name="s")

    def scalar_fn(x_hbm, table_hbm, shared_hbm, out_hbm,
                  idx_s, stage, idx_v, rows_v, shared_v,
                  sem_sh, sem_tb, sem_so, sem_vg):
        del idx_v, rows_v, shared_v, sem_vg
        cid = lax.axis_index("c")
        base = cid * bpc

        for c in range(nch):
            pltpu.sync_copy(x_hbm.at[pl.ds(base + c * _CHUNK, _CHUNK)], idx_s)

            def row(i, carry):
                r = idx_s[i]
                o = c * _CHUNK + i
                pltpu.async_copy(
                    shared_hbm, stage.at[o, pl.ds(0, _SHARED)], sem_sh
                )
                pltpu.async_copy(
                    table_hbm.at[r, pl.ds(_SHARED, _REST)],
                    stage.at[o, pl.ds(_SHARED, _REST)],
                    sem_tb,
                )
                return carry

            lax.fori_loop(0, _CHUNK, row, 0)

        def drain(i, carry):
            pltpu.make_async_copy(
                shared_hbm, stage.at[i, pl.ds(0, _SHARED)], sem_sh
            ).wait()
            pltpu.make_async_copy(
                table_hbm.at[0, pl.ds(_SHARED, _REST)],
                stage.at[i, pl.ds(_SHARED, _REST)],
                sem_tb,
            ).wait()
            return carry

        lax.fori_loop(0, bpc, drain, 0)
        pltpu.async_copy(stage, out_hbm.at[pl.ds(base, bpc)], sem_so).wait()

    def vector_fn(x_hbm, table_hbm, shared_hbm, out_hbm,
                  idx_v, stage, idx_tv, rows_v, shared_v,
                  sem_sh, sem_tb, sem_so, sem_vg):
        del idx_v, stage, sem_sh, sem_tb, sem_so
        wid = lax.axis_index("s") * nc + lax.axis_index("c")
        base = _SROWS + wid * bpw
        pltpu.sync_copy(x_hbm.at[pl.ds(base, bpw)], idx_tv)
        pltpu.sync_copy(shared_hbm, shared_v)

        def fire(g, carry):
            b0 = g * 16
            vi = idx_tv[pl.ds(b0, 16)]
            for j in range(16):
                r = vi[j]
                pltpu.async_copy(
                    table_hbm.at[pl.ds(r, 1), pl.ds(_SHARED, _REST)],
                    rows_v.at[pl.ds(b0 + j, 1), pl.ds(_SHARED, _REST)],
                    sem_vg,
                )
            return carry

        lax.fori_loop(0, bpw // 16, fire, 0)

        svec = shared_v[...]

        def prefill(i, carry):
            rows_v[i, pl.ds(0, _SHARED)] = svec
            return carry

        lax.fori_loop(0, bpw, prefill, 0)

        def drain(i, carry):
            pltpu.make_async_copy(
                table_hbm.at[pl.ds(0, 1), pl.ds(_SHARED, _REST)],
                rows_v.at[pl.ds(i, 1), pl.ds(_SHARED, _REST)],
                sem_vg,
            ).wait()
            return carry

        lax.fori_loop(0, bpw, drain, 0)
        pltpu.sync_copy(rows_v, out_hbm.at[pl.ds(base, bpw)])

    call = mpmd.mpmd_map(
        [(smesh, scalar_fn), (vmesh, vector_fn)],
        out_types=[jax.ShapeDtypeStruct((_B, _D), jnp.float32)],
        scratch_types=[
            pltpu.SMEM((_CHUNK,), jnp.int32) @ smesh,
            pltpu.VMEM_SHARED((bpc, _D), jnp.float32),
            pltpu.VMEM((bpw,), jnp.int32) @ vmesh,
            pltpu.VMEM((bpw, _D), jnp.float32) @ vmesh,
            pltpu.VMEM((_SHARED,), jnp.float32) @ vmesh,
            pltpu.SemaphoreType.DMA(()) @ smesh,
            pltpu.SemaphoreType.DMA(()) @ smesh,
            pltpu.SemaphoreType.DMA(()) @ smesh,
            pltpu.SemaphoreType.DMA(()) @ vmesh,
        ],
    )

    def run(X, table, shared):
        out = call(X, table, shared)
        return out[0] if isinstance(out, (list, tuple)) else out

    return run


def kernel(X, table, shared_embed):
    return _build()(X, table, shared_embed.reshape(_SHARED))


# speedup vs baseline: 1.5426x; 1.1871x over previous
import functools

import jax
import jax.numpy as jnp
from jax import lax
from jax.experimental import pallas as pl
from jax.experimental.pallas import tpu as pltpu
from jax.experimental.pallas import tpu_sc as plsc

_B = 16384
_D = 64
_SHARED = 16
_REST = _D - _SHARED
_NSEM = 8


@functools.cache
def _build():
    try:
        info = plsc.get_sparse_core_info()
        nc, ns = info.num_cores, info.num_subcores
    except Exception:
        nc, ns = 2, 16
    nw = nc * ns
    bpw = _B // nw
    mesh = plsc.VectorSubcoreMesh(core_axis_name="c", subcore_axis_name="s")

    @functools.partial(
        pl.kernel,
        mesh=mesh,
        out_type=jax.ShapeDtypeStruct((_B, _D), jnp.float32),
        scratch_types=[
            pltpu.VMEM((bpw,), jnp.int32),
            pltpu.VMEM((bpw, _D), jnp.float32),
            pltpu.VMEM((_SHARED,), jnp.float32),
        ]
        + [pltpu.SemaphoreType.DMA] * _NSEM,
    )
    def gather_kernel(x_hbm, table_hbm, shared_hbm, out_hbm,
                      idx_v, rows_v, shared_v, *sems):
        wid = lax.axis_index("s") * nc + lax.axis_index("c")
        base = wid * bpw
        pltpu.sync_copy(x_hbm.at[pl.ds(base, bpw)], idx_v)
        pltpu.sync_copy(shared_hbm, shared_v)

        def fire(g, carry):
            b0 = g * 16
            vi = idx_v[pl.ds(b0, 16)]
            for j in range(16):
                r = vi[j]
                pltpu.async_copy(
                    table_hbm.at[pl.ds(r, 1), pl.ds(_SHARED, _REST)],
                    rows_v.at[pl.ds(b0 + j, 1), pl.ds(_SHARED, _REST)],
                    sems[j % _NSEM],
                )
            return carry

        lax.fori_loop(0, bpw // 16, fire, 0)

        svec = shared_v[...]

        def prefill(i, carry):
            rows_v[i, pl.ds(0, _SHARED)] = svec
            return carry

        lax.fori_loop(0, bpw, prefill, 0)

        def drain(g, carry):
            for j in range(16):
                pltpu.make_async_copy(
                    table_hbm.at[pl.ds(0, 1), pl.ds(_SHARED, _REST)],
                    rows_v.at[pl.ds(g * 16 + j, 1), pl.ds(_SHARED, _REST)],
                    sems[j % _NSEM],
                ).wait()
            return carry

        lax.fori_loop(0, bpw // 16, drain, 0)
        pltpu.sync_copy(rows_v, out_hbm.at[pl.ds(base, bpw)])

    return gather_kernel


def kernel(X, table, shared_embed):
    return _build()(X, table, shared_embed.reshape(_SHARED))
